# baseline (speedup 1.0000x reference)
import functools

import jax
import jax.numpy as jnp
from jax import lax
from jax.experimental import pallas as pl
from jax.experimental.pallas import tpu as pltpu
from jax.experimental.pallas import tpu_sc as plsc

_N_TOK = 8192
_N_EMB = 8192
_DIM = 32
_TM = 256
_NB = _N_TOK // _TM

_SC_CORES = 2
_SC_SUBCORES = 16
_NW = _SC_CORES * _SC_SUBCORES
_B_PER_W = _N_TOK // _NW
_PAD_D = 128
_HROW = 16


_NCHUNK = _B_PER_W // 128


def _sc_body(idx_hbm, table_hbm, zeros_hbm, ones_hbm,
             zq_hbm, hist0_hbm, hist1_hbm,
             idx_v, idx2_v, rows_v, ones_v, sem, hist_sh):
    c = lax.axis_index("c")
    s = lax.axis_index("s")
    wid = s * _SC_CORES + c
    base = wid * _B_PER_W

    pltpu.sync_copy(idx_hbm.at[pl.ds(base, _B_PER_W)], idx_v)
    for j in range(_NCHUNK):
        pltpu.sync_copy(idx_hbm.at[pl.ds(base + j * 128, 128)], idx2_v.at[j])
    pltpu.sync_copy(ones_hbm, ones_v)
    pltpu.async_copy(table_hbm.at[idx_v], rows_v, sem).wait()
    pltpu.sync_copy(rows_v, zq_hbm.at[pl.ds(base, _B_PER_W)])

    @pl.when(s == 0)
    def _zero():
        pltpu.sync_copy(zeros_hbm, hist_sh)

    plsc.subcore_barrier()
    for j in range(_NCHUNK):
        pltpu.sync_copy(ones_v, hist_sh.at[idx2_v.at[j]], add=True)
    plsc.subcore_barrier()

    @pl.when((s == 0) & (c == 0))
    def _flush0():
        pltpu.sync_copy(hist_sh, hist0_hbm)

    @pl.when((s == 0) & (c == 1))
    def _flush1():
        pltpu.sync_copy(hist_sh, hist1_hbm)


def _sc_gather_hist(idx, table_padded, zeros, ones):
    k = functools.partial(
        pl.kernel,
        mesh=plsc.VectorSubcoreMesh(core_axis_name="c", subcore_axis_name="s"),
        out_type=[
            jax.ShapeDtypeStruct((_N_TOK, _PAD_D), jnp.float32),
            jax.ShapeDtypeStruct((_N_EMB, _HROW), jnp.float32),
            jax.ShapeDtypeStruct((_N_EMB, _HROW), jnp.float32),
        ],
        scratch_types=[
            pltpu.VMEM((_B_PER_W,), jnp.int32),
            pltpu.VMEM((_NCHUNK, 128), jnp.int32),
            pltpu.VMEM((_B_PER_W, _PAD_D), jnp.float32),
            pltpu.VMEM((128, _HROW), jnp.float32),
            pltpu.SemaphoreType.DMA,
            pltpu.VMEM_SHARED((_N_EMB, _HROW), jnp.float32),
        ],
    )(_sc_body)
    return k(idx, table_padded, zeros, ones)


def _vq_kernel(z_ref, w_ref, z2_ref, w2_ref,
               dist_ref, idx_ref, loss_ref, minsum_ref):
    i = pl.program_id(0)
    z = z_ref[...]
    w = w_ref[...]
    mm = jax.lax.dot_general(
        z, w, (((1,), (1,)), ((), ())),
        preferred_element_type=jnp.float32)
    dist = (z2_ref[...] + w2_ref[...]) - 2.0 * mm
    dist_ref[...] = dist

    minval = jnp.min(dist, axis=1, keepdims=True)
    iota = jax.lax.broadcasted_iota(jnp.int32, dist.shape, 1)
    big = jnp.int32(2 ** 30)
    idx = jnp.min(jnp.where(dist == minval, iota, big), axis=1)
    idx_ref[0, 0, :] = idx

    @pl.when(i == 0)
    def _init():
        minsum_ref[0, 0] = 0.0

    minsum_ref[0, 0] += jnp.sum(minval)

    @pl.when(i == _NB - 1)
    def _finish():
        loss = 1.25 * minsum_ref[0, 0] / float(_N_TOK * _DIM)
        loss_ref[...] = jnp.reshape(loss, (1, 1))


def _perp_kernel(hist_ref, perp_ref):
    h = hist_ref[0:1, :] + hist_ref[1:2, :]
    avg = h * (1.0 / float(_N_TOK))
    ent = -jnp.sum(avg * jnp.log(avg + 1e-10)) / float(_HROW)
    perp_ref[...] = jnp.reshape(jnp.exp(ent), (1, 1))


@functools.partial(jax.jit, static_argnames=())
def kernel(z_e, W):
    z_perm = jnp.transpose(z_e, (0, 2, 3, 1))
    flat = z_perm.reshape(-1, _DIM)
    z2 = jnp.sum(flat ** 2, axis=1, keepdims=True)
    w2 = jnp.sum(W ** 2, axis=1)

    dist, idx3, loss11 = pl.pallas_call(
        _vq_kernel,
        grid=(_NB,),
        in_specs=[
            pl.BlockSpec((_TM, _DIM), lambda i: (i, 0)),
            pl.BlockSpec((_N_EMB, _DIM), lambda i: (0, 0)),
            pl.BlockSpec((_TM, 1), lambda i: (i, 0)),
            pl.BlockSpec((1, _N_EMB), lambda i: (0, 0)),
        ],
        out_specs=[
            pl.BlockSpec((_TM, _N_EMB), lambda i: (i, 0)),
            pl.BlockSpec((1, 1, _TM), lambda i: (i, 0, 0)),
            pl.BlockSpec((1, 1), lambda i: (0, 0)),
        ],
        out_shape=[
            jax.ShapeDtypeStruct((_N_TOK, _N_EMB), jnp.float32),
            jax.ShapeDtypeStruct((_NB, 1, _TM), jnp.int32),
            jax.ShapeDtypeStruct((1, 1), jnp.float32),
        ],
        scratch_shapes=[
            pltpu.SMEM((1, 1), jnp.float32),
        ],
    )(flat, W, z2, w2.reshape(1, -1))

    idx = idx3.reshape(-1)
    w_pad = jnp.pad(W, ((0, 0), (0, _PAD_D - _DIM)))
    zeros = jnp.zeros((_N_EMB, _HROW), jnp.float32)
    ones = jnp.ones((128, _HROW), jnp.float32)
    zq_pad, hist0, hist1 = _sc_gather_hist(idx, w_pad, zeros, ones)
    zq = zq_pad[:, :_DIM]
    hist = jnp.stack([hist0, hist1])

    perp11 = pl.pallas_call(
        _perp_kernel,
        grid=(1,),
        in_specs=[pl.BlockSpec((_SC_CORES, _N_EMB * _HROW), lambda i: (0, 0))],
        out_specs=pl.BlockSpec((1, 1), lambda i: (0, 0)),
        out_shape=jax.ShapeDtypeStruct((1, 1), jnp.float32),
    )(hist.reshape(_SC_CORES, -1))

    z_q_out = jnp.transpose(zq.reshape(z_perm.shape), (0, 3, 1, 2))
    return (z_q_out, loss11[0, 0], perp11[0, 0], idx, dist)


# speedup vs baseline: 1.0173x; 1.0173x over previous
import functools

import jax
import jax.numpy as jnp
from jax import lax
from jax.experimental import pallas as pl
from jax.experimental.pallas import tpu as pltpu
from jax.experimental.pallas import tpu_sc as plsc

_N_TOK = 8192
_N_EMB = 8192
_DIM = 32
_TM = 512
_NB = _N_TOK // _TM

_SC_CORES = 2
_SC_SUBCORES = 16
_NW = _SC_CORES * _SC_SUBCORES
_B_PER_W = _N_TOK // _NW
_PAD_D = 128
_HROW = 16


_NCHUNK = _B_PER_W // 128


def _sc_body(idx_hbm, table_hbm, zeros_hbm, ones_hbm,
             zq_hbm, hist0_hbm, hist1_hbm,
             idx_v, idx2_v, rows_v, ones_v, sem, hist_sh):
    c = lax.axis_index("c")
    s = lax.axis_index("s")
    wid = s * _SC_CORES + c
    base = wid * _B_PER_W

    pltpu.sync_copy(idx_hbm.at[pl.ds(base, _B_PER_W)], idx_v)
    for j in range(_NCHUNK):
        pltpu.sync_copy(idx_hbm.at[pl.ds(base + j * 128, 128)], idx2_v.at[j])
    pltpu.sync_copy(ones_hbm, ones_v)
    pltpu.async_copy(table_hbm.at[idx_v], rows_v, sem).wait()
    pltpu.sync_copy(rows_v, zq_hbm.at[pl.ds(base, _B_PER_W)])

    @pl.when(s == 0)
    def _zero():
        pltpu.sync_copy(zeros_hbm, hist_sh)

    plsc.subcore_barrier()
    for j in range(_NCHUNK):
        pltpu.sync_copy(ones_v, hist_sh.at[idx2_v.at[j]], add=True)
    plsc.subcore_barrier()

    @pl.when((s == 0) & (c == 0))
    def _flush0():
        pltpu.sync_copy(hist_sh, hist0_hbm)

    @pl.when((s == 0) & (c == 1))
    def _flush1():
        pltpu.sync_copy(hist_sh, hist1_hbm)


def _sc_gather_hist(idx, table_padded, zeros, ones):
    k = functools.partial(
        pl.kernel,
        mesh=plsc.VectorSubcoreMesh(core_axis_name="c", subcore_axis_name="s"),
        out_type=[
            jax.ShapeDtypeStruct((_N_TOK, _PAD_D), jnp.float32),
            jax.ShapeDtypeStruct((_N_EMB, _HROW), jnp.float32),
            jax.ShapeDtypeStruct((_N_EMB, _HROW), jnp.float32),
        ],
        scratch_types=[
            pltpu.VMEM((_B_PER_W,), jnp.int32),
            pltpu.VMEM((_NCHUNK, 128), jnp.int32),
            pltpu.VMEM((_B_PER_W, _PAD_D), jnp.float32),
            pltpu.VMEM((128, _HROW), jnp.float32),
            pltpu.SemaphoreType.DMA,
            pltpu.VMEM_SHARED((_N_EMB, _HROW), jnp.float32),
        ],
    )(_sc_body)
    return k(idx, table_padded, zeros, ones)


def _vq_kernel(z_ref, w_ref, z2_ref, w2_ref,
               dist_ref, idx_ref, loss_ref, minsum_ref):
    i = pl.program_id(0)
    z = z_ref[...]
    w = w_ref[...]
    mm = jax.lax.dot_general(
        z, w, (((1,), (1,)), ((), ())),
        preferred_element_type=jnp.float32)
    dist = (z2_ref[...] + w2_ref[...]) - 2.0 * mm
    dist_ref[...] = dist

    minval = jnp.min(dist, axis=1, keepdims=True)
    iota = jax.lax.broadcasted_iota(jnp.int32, dist.shape, 1)
    big = jnp.int32(2 ** 30)
    idx = jnp.min(jnp.where(dist == minval, iota, big), axis=1)
    idx_ref[0, 0, :] = idx

    @pl.when(i == 0)
    def _init():
        minsum_ref[0, 0] = 0.0

    minsum_ref[0, 0] += jnp.sum(minval)

    @pl.when(i == _NB - 1)
    def _finish():
        loss = 1.25 * minsum_ref[0, 0] / float(_N_TOK * _DIM)
        loss_ref[...] = jnp.reshape(loss, (1, 1))


def _perp_kernel(hist_ref, perp_ref):
    h = hist_ref[0:1, :] + hist_ref[1:2, :]
    avg = h * (1.0 / float(_N_TOK))
    ent = -jnp.sum(avg * jnp.log(avg + 1e-10)) / float(_HROW)
    perp_ref[...] = jnp.reshape(jnp.exp(ent), (1, 1))


@functools.partial(jax.jit, static_argnames=())
def kernel(z_e, W):
    z_perm = jnp.transpose(z_e, (0, 2, 3, 1))
    flat = z_perm.reshape(-1, _DIM)
    z2 = jnp.sum(flat ** 2, axis=1, keepdims=True)
    w2 = jnp.sum(W ** 2, axis=1)

    dist, idx3, loss11 = pl.pallas_call(
        _vq_kernel,
        grid=(_NB,),
        in_specs=[
            pl.BlockSpec((_TM, _DIM), lambda i: (i, 0)),
            pl.BlockSpec((_N_EMB, _DIM), lambda i: (0, 0)),
            pl.BlockSpec((_TM, 1), lambda i: (i, 0)),
            pl.BlockSpec((1, _N_EMB), lambda i: (0, 0)),
        ],
        out_specs=[
            pl.BlockSpec((_TM, _N_EMB), lambda i: (i, 0)),
            pl.BlockSpec((1, 1, _TM), lambda i: (i, 0, 0)),
            pl.BlockSpec((1, 1), lambda i: (0, 0)),
        ],
        out_shape=[
            jax.ShapeDtypeStruct((_N_TOK, _N_EMB), jnp.float32),
            jax.ShapeDtypeStruct((_NB, 1, _TM), jnp.int32),
            jax.ShapeDtypeStruct((1, 1), jnp.float32),
        ],
        scratch_shapes=[
            pltpu.SMEM((1, 1), jnp.float32),
        ],
    )(flat, W, z2, w2.reshape(1, -1))

    idx = idx3.reshape(-1)
    w_pad = jnp.pad(W, ((0, 0), (0, _PAD_D - _DIM)))
    zeros = jnp.zeros((_N_EMB, _HROW), jnp.float32)
    ones = jnp.ones((128, _HROW), jnp.float32)
    zq_pad, hist0, hist1 = _sc_gather_hist(idx, w_pad, zeros, ones)
    zq = zq_pad[:, :_DIM]
    hist = jnp.stack([hist0, hist1])

    perp11 = pl.pallas_call(
        _perp_kernel,
        grid=(1,),
        in_specs=[pl.BlockSpec((_SC_CORES, _N_EMB * _HROW), lambda i: (0, 0))],
        out_specs=pl.BlockSpec((1, 1), lambda i: (0, 0)),
        out_shape=jax.ShapeDtypeStruct((1, 1), jnp.float32),
    )(hist.reshape(_SC_CORES, -1))

    z_q_out = jnp.transpose(zq.reshape(z_perm.shape), (0, 3, 1, 2))
    return (z_q_out, loss11[0, 0], perp11[0, 0], idx, dist)


# speedup vs baseline: 1.0218x; 1.0045x over previous
import functools

import jax
import jax.numpy as jnp
from jax import lax
from jax.experimental import pallas as pl
from jax.experimental.pallas import tpu as pltpu
from jax.experimental.pallas import tpu_sc as plsc

_N_TOK = 8192
_N_EMB = 8192
_DIM = 32
_TM = 512
_NB = _N_TOK // _TM

_SC_CORES = 2
_SC_SUBCORES = 16
_NW = _SC_CORES * _SC_SUBCORES
_B_PER_W = _N_TOK // _NW


_PAD_D = 128


def _sc_gather_body(idx_hbm, table_hbm, out_hbm, idx_v, rows_v, sem):
    wid = lax.axis_index("s") * _SC_CORES + lax.axis_index("c")
    base = wid * _B_PER_W
    pltpu.sync_copy(idx_hbm.at[pl.ds(base, _B_PER_W)], idx_v)
    pltpu.async_copy(table_hbm.at[idx_v], rows_v, sem).wait()
    pltpu.sync_copy(rows_v, out_hbm.at[pl.ds(base, _B_PER_W)])


def _sc_gather(idx, table_padded):
    k = functools.partial(
        pl.kernel,
        mesh=plsc.VectorSubcoreMesh(core_axis_name="c", subcore_axis_name="s"),
        out_type=jax.ShapeDtypeStruct((_N_TOK, _PAD_D), jnp.float32),
        scratch_types=[
            pltpu.VMEM((_B_PER_W,), jnp.int32),
            pltpu.VMEM((_B_PER_W, _PAD_D), jnp.float32),
            pltpu.SemaphoreType.DMA,
        ],
    )(_sc_gather_body)
    return k(idx, table_padded)


def _vq_kernel(z_ref, w_ref, z2_ref, w2_ref,
               dist_ref, idx_ref, loss_ref, perp_ref,
               counts_ref, minsum_ref):
    i = pl.program_id(0)
    z = z_ref[...]
    w = w_ref[...]
    mm = jax.lax.dot_general(
        z, w, (((1,), (1,)), ((), ())),
        preferred_element_type=jnp.float32)
    dist = (z2_ref[...] + w2_ref[...]) - 2.0 * mm
    dist_ref[...] = dist

    minval = jnp.min(dist, axis=1, keepdims=True)
    iota = jax.lax.broadcasted_iota(jnp.int32, dist.shape, 1)
    big = jnp.int32(2 ** 30)
    idx = jnp.min(jnp.where(dist == minval, iota, big), axis=1)
    idx_ref[0, 0, :] = idx

    onehot = (iota == idx[:, None]).astype(jnp.float32)

    @pl.when(i == 0)
    def _init():
        counts_ref[...] = jnp.zeros_like(counts_ref)
        minsum_ref[0, 0] = 0.0

    counts_ref[...] += jnp.sum(onehot, axis=0, keepdims=True)
    minsum_ref[0, 0] += jnp.sum(minval)

    @pl.when(i == _NB - 1)
    def _finish():
        loss = 1.25 * minsum_ref[0, 0] / float(_N_TOK * _DIM)
        loss_ref[...] = jnp.reshape(loss, (1, 1))
        avg = counts_ref[...] / float(_N_TOK)
        ent = -jnp.sum(avg * jnp.log(avg + 1e-10))
        perp_ref[...] = jnp.reshape(jnp.exp(ent), (1, 1))


@functools.partial(jax.jit, static_argnames=())
def kernel(z_e, W):
    z_perm = jnp.transpose(z_e, (0, 2, 3, 1))
    flat = z_perm.reshape(-1, _DIM)
    z2 = jnp.sum(flat ** 2, axis=1, keepdims=True)
    w2 = jnp.sum(W ** 2, axis=1)

    dist, idx3, loss11, perp11 = pl.pallas_call(
        _vq_kernel,
        grid=(_NB,),
        in_specs=[
            pl.BlockSpec((_TM, _DIM), lambda i: (i, 0)),
            pl.BlockSpec((_N_EMB, _DIM), lambda i: (0, 0)),
            pl.BlockSpec((_TM, 1), lambda i: (i, 0)),
            pl.BlockSpec((1, _N_EMB), lambda i: (0, 0)),
        ],
        out_specs=[
            pl.BlockSpec((_TM, _N_EMB), lambda i: (i, 0)),
            pl.BlockSpec((1, 1, _TM), lambda i: (i, 0, 0)),
            pl.BlockSpec((1, 1), lambda i: (0, 0)),
            pl.BlockSpec((1, 1), lambda i: (0, 0)),
        ],
        out_shape=[
            jax.ShapeDtypeStruct((_N_TOK, _N_EMB), jnp.float32),
            jax.ShapeDtypeStruct((_NB, 1, _TM), jnp.int32),
            jax.ShapeDtypeStruct((1, 1), jnp.float32),
            jax.ShapeDtypeStruct((1, 1), jnp.float32),
        ],
        scratch_shapes=[
            pltpu.VMEM((1, _N_EMB), jnp.float32),
            pltpu.SMEM((1, 1), jnp.float32),
        ],
    )(flat, W, z2, w2.reshape(1, -1))

    idx = idx3.reshape(-1)
    w_pad = jnp.pad(W, ((0, 0), (0, _PAD_D - _DIM)))
    zq = _sc_gather(idx, w_pad)[:, :_DIM]
    z_q_out = jnp.transpose(zq.reshape(z_perm.shape), (0, 3, 1, 2))
    return (z_q_out, loss11[0, 0], perp11[0, 0], idx, dist)


# speedup vs baseline: 1.0431x; 1.0208x over previous
import functools

import jax
import jax.numpy as jnp
from jax import lax
from jax.experimental import pallas as pl
from jax.experimental.pallas import tpu as pltpu
from jax.experimental.pallas import tpu_sc as plsc

_N_TOK = 8192
_N_EMB = 8192
_DIM = 32
_TM = 512
_NB = _N_TOK // _TM

_SC_CORES = 2
_SC_SUBCORES = 16
_NW = _SC_CORES * _SC_SUBCORES
_B_PER_W = _N_TOK // _NW


_PAD_D = 128


def _sc_gather_body(idx_hbm, table_hbm, out_hbm, idx_v, rows_v, sem):
    wid = lax.axis_index("s") * _SC_CORES + lax.axis_index("c")
    base = wid * _B_PER_W
    pltpu.sync_copy(idx_hbm.at[pl.ds(base, _B_PER_W)], idx_v)
    pltpu.async_copy(table_hbm.at[idx_v], rows_v, sem).wait()
    pltpu.sync_copy(rows_v, out_hbm.at[pl.ds(base, _B_PER_W)])


def _sc_gather(idx, table_padded):
    k = functools.partial(
        pl.kernel,
        mesh=plsc.VectorSubcoreMesh(core_axis_name="c", subcore_axis_name="s"),
        out_type=jax.ShapeDtypeStruct((_N_TOK, _PAD_D), jnp.float32),
        scratch_types=[
            pltpu.VMEM((_B_PER_W,), jnp.int32),
            pltpu.VMEM((_B_PER_W, _PAD_D), jnp.float32),
            pltpu.SemaphoreType.DMA,
        ],
    )(_sc_gather_body)
    return k(idx, table_padded)


def _vq_kernel(z_ref, w_ref, z2_ref, w2_ref,
               dist_ref, idx_ref, loss_ref, perp_ref,
               counts_ref, minsum_ref):
    i = pl.program_id(0)
    zb = z_ref[...]
    z = jnp.transpose(zb.reshape(_DIM, _TM), (1, 0))
    w = w_ref[...]
    mm = jax.lax.dot_general(
        z, w, (((1,), (1,)), ((), ())),
        preferred_element_type=jnp.float32)
    dist = (z2_ref[...] + w2_ref[...]) - 2.0 * mm
    dist_ref[...] = dist

    minval = jnp.min(dist, axis=1, keepdims=True)
    iota = jax.lax.broadcasted_iota(jnp.int32, dist.shape, 1)
    big = jnp.int32(2 ** 30)
    idx = jnp.min(jnp.where(dist == minval, iota, big), axis=1)
    idx_ref[0, 0, :] = idx

    onehot = (iota == idx[:, None]).astype(jnp.float32)

    @pl.when(i == 0)
    def _init():
        counts_ref[...] = jnp.zeros_like(counts_ref)
        minsum_ref[0, 0] = 0.0

    counts_ref[...] += jnp.sum(onehot, axis=0, keepdims=True)
    minsum_ref[0, 0] += jnp.sum(minval)

    @pl.when(i == _NB - 1)
    def _finish():
        loss = 1.25 * minsum_ref[0, 0] / float(_N_TOK * _DIM)
        loss_ref[...] = jnp.reshape(loss, (1, 1))
        avg = counts_ref[...] / float(_N_TOK)
        ent = -jnp.sum(avg * jnp.log(avg + 1e-10))
        perp_ref[...] = jnp.reshape(jnp.exp(ent), (1, 1))


@functools.partial(jax.jit, static_argnames=())
def kernel(z_e, W):
    z2 = jnp.sum(z_e ** 2, axis=1).reshape(-1, 1)
    w2 = jnp.sum(W ** 2, axis=1)
    _hrows = _TM // 32

    dist, idx3, loss11, perp11 = pl.pallas_call(
        _vq_kernel,
        grid=(_NB,),
        in_specs=[
            pl.BlockSpec((1, _DIM, _hrows, 32),
                         lambda i: (i // (32 // _hrows), 0, i % (32 // _hrows), 0)),
            pl.BlockSpec((_N_EMB, _DIM), lambda i: (0, 0)),
            pl.BlockSpec((_TM, 1), lambda i: (i, 0)),
            pl.BlockSpec((1, _N_EMB), lambda i: (0, 0)),
        ],
        out_specs=[
            pl.BlockSpec((_TM, _N_EMB), lambda i: (i, 0)),
            pl.BlockSpec((1, 1, _TM), lambda i: (i, 0, 0)),
            pl.BlockSpec((1, 1), lambda i: (0, 0)),
            pl.BlockSpec((1, 1), lambda i: (0, 0)),
        ],
        out_shape=[
            jax.ShapeDtypeStruct((_N_TOK, _N_EMB), jnp.float32),
            jax.ShapeDtypeStruct((_NB, 1, _TM), jnp.int32),
            jax.ShapeDtypeStruct((1, 1), jnp.float32),
            jax.ShapeDtypeStruct((1, 1), jnp.float32),
        ],
        scratch_shapes=[
            pltpu.VMEM((1, _N_EMB), jnp.float32),
            pltpu.SMEM((1, 1), jnp.float32),
        ],
    )(z_e, W, z2, w2.reshape(1, -1))

    idx = idx3.reshape(-1)
    w_pad = jnp.pad(W, ((0, 0), (0, _PAD_D - _DIM)))
    zq = _sc_gather(idx, w_pad)[:, :_DIM]
    z_q_out = jnp.transpose(zq.reshape(8, 32, 32, _DIM), (0, 3, 1, 2))
    return (z_q_out, loss11[0, 0], perp11[0, 0], idx, dist)
